# Initial kernel scaffold; baseline (speedup 1.0000x reference)
import functools

import jax
import jax.numpy as jnp
from jax import lax
from jax.experimental import pallas as pl
from jax.experimental.pallas import tpu as pltpu
from jax.experimental.pallas import tpu_sc as plsc

F32 = jnp.float32

N = 10000
E = 320000
D = 128
G = 64

NC = 2
NS = 16
NW = NC * NS
EPW = E // NW
RPT = N // NS
CH = 80
NCH = EPW // CH
DEGR = (N + 15) // 16


def _build_deg(interpret=False):
    mesh = plsc.VectorSubcoreMesh(core_axis_name="c", subcore_axis_name="s")

    @functools.partial(
        pl.kernel,
        out_type=jax.ShapeDtypeStruct((NC, DEGR, 16), F32),
        mesh=mesh,
        interpret=interpret,
        scratch_types=[
            pltpu.VMEM((EPW,), jnp.int32),
            pltpu.VMEM((5, 125), jnp.int32),
            pltpu.VMEM((DEGR, 16), F32),
            pltpu.VMEM_SHARED((DEGR, 16), F32),
        ],
    )
    def deg_k(dst_hbm, rowidx_hbm, out_hbm, dst_v, rowidx_v, deg_priv, deg_sh):
        cid = lax.axis_index("c")
        sid = lax.axis_index("s")
        wid = sid * NC + cid

        zero16 = jnp.zeros((16,), F32)

        def zbody(i, _):
            deg_priv[i, :] = zero16
            return 0

        lax.fori_loop(0, DEGR, zbody, 0)

        @pl.when(sid == 0)
        def _():
            pltpu.sync_copy(deg_priv, deg_sh)

        plsc.subcore_barrier()

        pltpu.sync_copy(dst_hbm.at[wid], dst_v)
        pltpu.sync_copy(rowidx_hbm, rowidx_v)

        ones16 = jnp.ones((16,), F32)

        def body(i, _):
            d = dst_v[pl.ds(i * 16, 16)]
            plsc.addupdate_scatter(deg_priv, [d >> 4, d & 15], ones16)
            return 0

        lax.fori_loop(0, EPW // 16, body, 0)

        for k in range(5):
            pltpu.sync_copy(deg_priv.at[pl.ds(k * 125, 125)],
                            deg_sh.at[rowidx_v.at[k]], add=True)

        plsc.subcore_barrier()

        @pl.when(sid == 0)
        def _():
            pltpu.sync_copy(deg_sh, out_hbm.at[cid])

    return deg_k


def _normxn_body(deg_ref, x_ref, xn_ref, norm_ref):
    d2 = deg_ref[...]
    deg = d2[0] + d2[1]
    norm = jnp.where(deg > 0, lax.rsqrt(jnp.maximum(deg, 1.0)), 0.0)
    norm_ref[...] = norm
    xn_ref[...] = x_ref[...] * norm


def _normxn_call(deg2, x, interpret=False):
    B = 2000
    grid = N // B
    return pl.pallas_call(
        _normxn_body,
        grid=(grid,),
        in_specs=[
            pl.BlockSpec((2, B, 1), lambda i: (0, i, 0)),
            pl.BlockSpec((B, D), lambda i: (i, 0)),
        ],
        out_specs=[
            pl.BlockSpec((B, D), lambda i: (i, 0)),
            pl.BlockSpec((B, 1), lambda i: (i, 0)),
        ],
        out_shape=[
            jax.ShapeDtypeStruct((N, D), F32),
            jax.ShapeDtypeStruct((N, 1), F32),
        ],
        interpret=interpret,
    )(deg2, x)


def _build_agg(interpret=False):
    mesh = plsc.VectorSubcoreMesh(core_axis_name="c", subcore_axis_name="s")

    @functools.partial(
        pl.kernel,
        out_type=jax.ShapeDtypeStruct((NC, N, D), F32),
        mesh=mesh,
        interpret=interpret,
        scratch_types=[
            pltpu.VMEM((NCH, CH), jnp.int32),
            pltpu.VMEM((NCH, CH), jnp.int32),
            pltpu.VMEM((CH, D), F32),
            pltpu.VMEM((125, D), F32),
            pltpu.VMEM_SHARED((N, D), F32),
            pltpu.SemaphoreType.DMA,
        ],
    )
    def agg_k(xn_hbm, src_hbm, dst_hbm, out_hbm,
              src_v, dst_v, rows_v, zbuf, acc_sh, sem):
        cid = lax.axis_index("c")
        sid = lax.axis_index("s")
        wid = sid * NC + cid

        zero16 = jnp.zeros((16,), F32)

        def zbody(i, _):
            for j in range(D // 16):
                zbuf[i, pl.ds(j * 16, 16)] = zero16
            return 0

        lax.fori_loop(0, 125, zbody, 0)

        for k in range(RPT // 125):
            pltpu.sync_copy(zbuf, acc_sh.at[pl.ds(sid * RPT + k * 125, 125)])

        plsc.subcore_barrier()

        pltpu.sync_copy(src_hbm.at[wid], src_v)
        pltpu.sync_copy(dst_hbm.at[wid], dst_v)

        def chunk(c, _):
            pltpu.async_copy(xn_hbm.at[src_v.at[c]], rows_v, sem).wait()
            pltpu.sync_copy(rows_v, acc_sh.at[dst_v.at[c]], add=True)
            return 0

        lax.fori_loop(0, NCH, chunk, 0)

        plsc.subcore_barrier()

        pltpu.sync_copy(acc_sh.at[pl.ds(sid * RPT, RPT)],
                        out_hbm.at[cid, pl.ds(sid * RPT, RPT)])

    return agg_k


_DB = 2000


def _elu(v):
    return jnp.where(v > 0, v, jnp.exp(v) - 1.0)


def _head_body(p_ref, norm_ref, batch_ref, Wg_ref, bg_ref, W0_ref, b0_ref,
               W1_ref, b1_ref, W2_ref, b2_ref, out_ref,
               acc_s, acc_m, acc_c, carry_v, carry_b):
    B = _DB
    i = pl.program_id(0)
    nsteps = pl.num_programs(0)

    @pl.when(i == 0)
    def _():
        acc_s[...] = jnp.zeros_like(acc_s)
        acc_m[...] = jnp.full_like(acc_m, -1e30)
        acc_c[...] = jnp.zeros_like(acc_c)
        carry_v[...] = jnp.full_like(carry_v, -1e30)
        carry_b[...] = jnp.full_like(carry_b, -1)

    p = p_ref[...]
    agg = (p[0] + p[1]) * norm_ref[...]
    h = _elu(agg @ Wg_ref[...] + bg_ref[...])

    bt = batch_ref[...]
    onehot = (bt == lax.broadcasted_iota(jnp.int32, (1, G), 1)).astype(F32)

    dn = (((0,), (0,)), ((), ()))
    acc_s[...] += lax.dot_general(onehot, h, dn, preferred_element_type=F32)
    acc_c[...] += lax.dot_general(onehot, jnp.ones((B, 1), F32), dn,
                                  preferred_element_type=F32)

    pm = jnp.where(bt == carry_b[...], jnp.maximum(h, carry_v[...]), h)
    sh = 1
    while sh < B:
        pm_s = jnp.concatenate(
            [jnp.full((sh, D), -1e30, F32), pm[:B - sh]], axis=0)
        bt_s = jnp.concatenate(
            [jnp.full((sh, 1), -1, jnp.int32), bt[:B - sh]], axis=0)
        pm = jnp.where(bt_s == bt, jnp.maximum(pm, pm_s), pm)
        sh *= 2

    bt_n = jnp.concatenate(
        [bt[1:], jnp.full((1, 1), -2, jnp.int32)], axis=0)
    endm = (bt != bt_n).astype(F32)
    m_part = lax.dot_general(onehot, pm * endm, dn, preferred_element_type=F32)
    g_part = lax.dot_general(onehot, endm, dn, preferred_element_type=F32)
    acc_m[...] = jnp.where(g_part > 0, jnp.maximum(acc_m[...], m_part),
                           acc_m[...])
    carry_v[...] = pm[B - 1:B, :]
    carry_b[...] = bt[B - 1:B, :]

    @pl.when(i == nsteps - 1)
    def _():
        cnt = acc_c[...]
        mean = acc_s[...] / jnp.maximum(cnt, 1.0)
        mx = jnp.where(cnt > 0, acc_m[...], 0.0)
        r = jnp.concatenate([mean, mx, acc_s[...]], axis=1)
        y = _elu(r @ W0_ref[...] + b0_ref[...])
        y = _elu(y @ W1_ref[...] + b1_ref[...])
        out_ref[...] = y @ W2_ref[...] + b2_ref[...]


def _head_call(partials, norm, batch2, W_g, b_g, W0, b0, W1, b1, W2, b2,
               interpret=False):
    B = _DB
    grid = N // B
    OUT_CH = W2.shape[1]

    def full(shape):
        return pl.BlockSpec(shape, lambda *_: tuple(0 for _ in shape))

    return pl.pallas_call(
        _head_body,
        grid=(grid,),
        in_specs=[
            pl.BlockSpec((2, B, D), lambda i: (0, i, 0)),
            pl.BlockSpec((B, 1), lambda i: (i, 0)),
            pl.BlockSpec((B, 1), lambda i: (i, 0)),
            full((D, D)), full((1, D)),
            full((3 * D, W0.shape[1])), full((1, W0.shape[1])),
            full((W1.shape[0], W1.shape[1])), full((1, W1.shape[1])),
            full((W2.shape[0], OUT_CH)), full((1, OUT_CH)),
        ],
        out_specs=pl.BlockSpec((G, OUT_CH), lambda i: (0, 0)),
        out_shape=jax.ShapeDtypeStruct((G, OUT_CH), F32),
        scratch_shapes=[
            pltpu.VMEM((G, D), F32),
            pltpu.VMEM((G, D), F32),
            pltpu.VMEM((G, 1), F32),
            pltpu.VMEM((1, D), F32),
            pltpu.VMEM((1, 1), jnp.int32),
        ],
        interpret=interpret,
    )(partials, norm, batch2, W_g, b_g, W0, b0, W1, b1, W2, b2)


def kernel(x, edge_index, batch, W_g, b_g, W0, b0, W1, b1, W2, b2):
    src = edge_index[0]
    dst = edge_index[1]

    deg2 = _build_deg()(dst.reshape(NW, EPW),
                        jnp.arange(DEGR, dtype=jnp.int32).reshape(5, 125))
    deg2 = deg2.reshape(2, N, 1)

    xn, norm = _normxn_call(deg2, x)

    partials = _build_agg()(xn, src.reshape(NW, NCH, CH),
                            dst.reshape(NW, NCH, CH))

    return _head_call(partials, norm, batch.reshape(N, 1),
                      W_g, b_g.reshape(1, D), W0, b0.reshape(1, -1),
                      W1, b1.reshape(1, -1), W2, b2.reshape(1, -1))


# speedup vs baseline: 16.9781x; 16.9781x over previous
import functools

import jax
import jax.numpy as jnp
from jax import lax
from jax.experimental import pallas as pl
from jax.experimental.pallas import tpu as pltpu
from jax.experimental.pallas import tpu_sc as plsc

F32 = jnp.float32

N = 10000
E = 320000
D = 128
G = 64

NC = 2
NS = 16
NW = NC * NS
EPW = E // NW
RPT = N // NS
CH = 80
NCH = EPW // CH
DEGR = (N + 127) // 128


def _build_deg(interpret=False):
    mesh = plsc.VectorSubcoreMesh(core_axis_name="c", subcore_axis_name="s")

    @functools.partial(
        pl.kernel,
        out_type=jax.ShapeDtypeStruct((NC, DEGR, 128), F32),
        mesh=mesh,
        interpret=interpret,
        compiler_params=pltpu.CompilerParams(needs_layout_passes=False, use_tc_tiling_on_sc=False),
        scratch_types=[
            pltpu.VMEM((EPW,), jnp.int32),
            pltpu.VMEM((1, DEGR), jnp.int32),
            pltpu.VMEM((DEGR * 128,), F32),
            pltpu.VMEM((DEGR, 128), F32),
            pltpu.VMEM_SHARED((DEGR, 128), F32),
        ],
    )
    def deg_k(dst_hbm, rowidx_hbm, out_hbm, dst_v, rowidx_v, deg_flat,
              deg_priv, deg_sh):
        cid = lax.axis_index("c")
        sid = lax.axis_index("s")
        wid = sid * NC + cid

        zero16 = jnp.zeros((16,), F32)

        def zbody2(i, _):
            deg_flat[pl.ds(i * 16, 16)] = zero16
            return 0

        def zbody3(i, _):
            for j in range(128 // 16):
                deg_priv[i, pl.ds(j * 16, 16)] = zero16
            return 0

        lax.fori_loop(0, DEGR * 8, zbody2, 0)
        lax.fori_loop(0, DEGR, zbody3, 0)

        @pl.when(sid == 0)
        def _():
            pltpu.sync_copy(deg_priv, deg_sh)

        plsc.subcore_barrier()

        pltpu.sync_copy(dst_hbm.at[wid], dst_v)
        pltpu.sync_copy(rowidx_hbm, rowidx_v)

        ones16 = jnp.ones((16,), F32)

        def body(i, _):
            d = dst_v[pl.ds(i * 16, 16)]
            plsc.addupdate_scatter(deg_flat, [d], ones16)
            return 0

        lax.fori_loop(0, EPW // 16, body, 0)

        def pack(i, _):
            for j in range(128 // 16):
                deg_priv[i, pl.ds(j * 16, 16)] = deg_flat[
                    pl.ds(i * 128 + j * 16, 16)]
            return 0

        lax.fori_loop(0, DEGR, pack, 0)

        pltpu.sync_copy(deg_priv, deg_sh.at[rowidx_v.at[0]], add=True)

        plsc.subcore_barrier()

        @pl.when(sid == 0)
        def _():
            pltpu.sync_copy(deg_sh, out_hbm.at[cid])

    return deg_k


def _normxn_body(deg_ref, x_ref, lo_ref, hi_ref, norm_ref):
    d2 = deg_ref[...]
    deg = d2[0] + d2[1]
    norm = jnp.where(deg > 0, lax.rsqrt(jnp.maximum(deg, 1.0)), 0.0)
    norm_ref[...] = norm
    xn = x_ref[...] * norm
    lo_ref[...] = xn[:, :D // 2]
    hi_ref[...] = xn[:, D // 2:]


def _normxn_call(deg2, x, interpret=False):
    B = 2000
    grid = N // B
    return pl.pallas_call(
        _normxn_body,
        grid=(grid,),
        in_specs=[
            pl.BlockSpec((2, B, 1), lambda i: (0, i, 0)),
            pl.BlockSpec((B, D), lambda i: (i, 0)),
        ],
        out_specs=[
            pl.BlockSpec((B, D // 2), lambda i: (i, 0)),
            pl.BlockSpec((B, D // 2), lambda i: (i, 0)),
            pl.BlockSpec((B, 1), lambda i: (i, 0)),
        ],
        out_shape=[
            jax.ShapeDtypeStruct((N, D // 2), F32),
            jax.ShapeDtypeStruct((N, D // 2), F32),
            jax.ShapeDtypeStruct((N, 1), F32),
        ],
        interpret=interpret,
    )(deg2, x)


def _build_agg(interpret=False):
    mesh = plsc.VectorSubcoreMesh(core_axis_name="c", subcore_axis_name="s")

    HD = D // 2

    @functools.partial(
        pl.kernel,
        out_type=(jax.ShapeDtypeStruct((NC, N, HD), F32),
                  jax.ShapeDtypeStruct((NC, N, HD), F32)),
        mesh=mesh,
        interpret=interpret,
        compiler_params=pltpu.CompilerParams(needs_layout_passes=False, use_tc_tiling_on_sc=False),
        scratch_types=[
            pltpu.VMEM((NCH, CH), jnp.int32),
            pltpu.VMEM((NCH, CH), jnp.int32),
            pltpu.VMEM((CH, HD), F32),
            pltpu.VMEM((80, HD), F32),
            pltpu.VMEM_SHARED((N, HD), F32),
            pltpu.SemaphoreType.DMA,
        ],
    )
    def agg_k(lo_hbm, hi_hbm, src_hbm, dst_hbm, outlo_hbm, outhi_hbm,
              src_v, dst_v, rows_v, zbuf, acc_sh, sem):
        cid = lax.axis_index("c")
        sid = lax.axis_index("s")
        wid = sid * NC + cid

        zero16 = jnp.zeros((16,), F32)

        def zbody(i, _):
            for j in range(HD // 16):
                zbuf[i, pl.ds(j * 16, 16)] = zero16
            return 0

        lax.fori_loop(0, 80, zbody, 0)

        pltpu.sync_copy(src_hbm.at[wid], src_v)
        pltpu.sync_copy(dst_hbm.at[wid], dst_v)

        NZ = N // 80

        for tab_hbm, out_hbm in ((lo_hbm, outlo_hbm), (hi_hbm, outhi_hbm)):
            for k in range((NZ + NS - 1) // NS):
                c = sid + NS * k

                @pl.when(c < NZ)
                def _(c=c):
                    off = pl.multiple_of(c * 80, 80)
                    pltpu.sync_copy(zbuf, acc_sh.at[pl.ds(off, 80)])

            plsc.subcore_barrier()

            def chunk(c, _):
                pltpu.async_copy(tab_hbm.at[src_v.at[c]], rows_v, sem).wait()
                pltpu.sync_copy(rows_v, acc_sh.at[dst_v.at[c]], add=True)
                return 0

            lax.fori_loop(0, NCH, chunk, 0)

            plsc.subcore_barrier()

            for k in range((NZ + NS - 1) // NS):
                c = sid + NS * k

                @pl.when(c < NZ)
                def _(c=c, out_hbm=out_hbm):
                    off = pl.multiple_of(c * 80, 80)
                    pltpu.sync_copy(acc_sh.at[pl.ds(off, 80)],
                                    out_hbm.at[cid, pl.ds(off, 80)])

            plsc.subcore_barrier()

    return agg_k


_DB = 2000


def _elu(v):
    return jnp.where(v > 0, v, jnp.exp(v) - 1.0)


def _head_body(plo_ref, phi_ref, norm_ref, batch_ref, Wg_ref, bg_ref,
               W0_ref, b0_ref, W1_ref, b1_ref, W2_ref, b2_ref, out_ref,
               acc_s, acc_m, acc_c, carry_v, carry_b):
    B = _DB
    i = pl.program_id(0)
    nsteps = pl.num_programs(0)

    @pl.when(i == 0)
    def _():
        acc_s[...] = jnp.zeros_like(acc_s)
        acc_m[...] = jnp.full_like(acc_m, -1e30)
        acc_c[...] = jnp.zeros_like(acc_c)
        carry_v[...] = jnp.full_like(carry_v, -1e30)
        carry_b[...] = jnp.full_like(carry_b, -1)

    plo = plo_ref[...]
    phi = phi_ref[...]
    agg = jnp.concatenate([plo[0] + plo[1], phi[0] + phi[1]],
                          axis=1) * norm_ref[...]
    h = _elu(agg @ Wg_ref[...] + bg_ref[...])

    bt = batch_ref[...]
    onehot = (bt == lax.broadcasted_iota(jnp.int32, (1, G), 1)).astype(F32)

    dn = (((0,), (0,)), ((), ()))
    acc_s[...] += lax.dot_general(onehot, h, dn, preferred_element_type=F32)
    acc_c[...] += lax.dot_general(onehot, jnp.ones((B, 1), F32), dn,
                                  preferred_element_type=F32)

    pm = jnp.where(bt == carry_b[...], jnp.maximum(h, carry_v[...]), h)
    sh = 1
    while sh < B:
        pm_s = jnp.concatenate(
            [jnp.full((sh, D), -1e30, F32), pm[:B - sh]], axis=0)
        bt_s = jnp.concatenate(
            [jnp.full((sh, 1), -1, jnp.int32), bt[:B - sh]], axis=0)
        pm = jnp.where(bt_s == bt, jnp.maximum(pm, pm_s), pm)
        sh *= 2

    bt_n = jnp.concatenate(
        [bt[1:], jnp.full((1, 1), -2, jnp.int32)], axis=0)
    endm = (bt != bt_n).astype(F32)
    m_part = lax.dot_general(onehot, pm * endm, dn, preferred_element_type=F32)
    g_part = lax.dot_general(onehot, endm, dn, preferred_element_type=F32)
    acc_m[...] = jnp.where(g_part > 0, jnp.maximum(acc_m[...], m_part),
                           acc_m[...])
    carry_v[...] = pm[B - 1:B, :]
    carry_b[...] = bt[B - 1:B, :]

    @pl.when(i == nsteps - 1)
    def _():
        cnt = acc_c[...]
        mean = acc_s[...] / jnp.maximum(cnt, 1.0)
        mx = jnp.where(cnt > 0, acc_m[...], 0.0)
        r = jnp.concatenate([mean, mx, acc_s[...]], axis=1)
        y = _elu(r @ W0_ref[...] + b0_ref[...])
        y = _elu(y @ W1_ref[...] + b1_ref[...])
        out_ref[...] = y @ W2_ref[...] + b2_ref[...]


def _head_call(p_lo, p_hi, norm, batch2, W_g, b_g, W0, b0, W1, b1, W2, b2,
               interpret=False):
    B = _DB
    grid = N // B
    OUT_CH = W2.shape[1]

    def full(shape):
        return pl.BlockSpec(shape, lambda *_: tuple(0 for _ in shape))

    return pl.pallas_call(
        _head_body,
        grid=(grid,),
        in_specs=[
            pl.BlockSpec((2, B, D // 2), lambda i: (0, i, 0)),
            pl.BlockSpec((2, B, D // 2), lambda i: (0, i, 0)),
            pl.BlockSpec((B, 1), lambda i: (i, 0)),
            pl.BlockSpec((B, 1), lambda i: (i, 0)),
            full((D, D)), full((1, D)),
            full((3 * D, W0.shape[1])), full((1, W0.shape[1])),
            full((W1.shape[0], W1.shape[1])), full((1, W1.shape[1])),
            full((W2.shape[0], OUT_CH)), full((1, OUT_CH)),
        ],
        out_specs=pl.BlockSpec((G, OUT_CH), lambda i: (0, 0)),
        out_shape=jax.ShapeDtypeStruct((G, OUT_CH), F32),
        scratch_shapes=[
            pltpu.VMEM((G, D), F32),
            pltpu.VMEM((G, D), F32),
            pltpu.VMEM((G, 1), F32),
            pltpu.VMEM((1, D), F32),
            pltpu.VMEM((1, 1), jnp.int32),
        ],
        interpret=interpret,
    )(p_lo, p_hi, norm, batch2, W_g, b_g, W0, b0, W1, b1, W2, b2)


def kernel(x, edge_index, batch, W_g, b_g, W0, b0, W1, b1, W2, b2):
    src = edge_index[0]
    dst = edge_index[1]

    deg2 = _build_deg()(dst.reshape(NW, EPW),
                        jnp.arange(DEGR, dtype=jnp.int32).reshape(1, DEGR))
    deg2 = deg2.reshape(2, DEGR * 128, 1)[:, :N]

    xn_lo, xn_hi, norm = _normxn_call(deg2, x)

    p_lo, p_hi = _build_agg()(xn_lo, xn_hi, src.reshape(NW, NCH, CH),
                              dst.reshape(NW, NCH, CH))

    return _head_call(p_lo, p_hi, norm, batch.reshape(N, 1),
                      W_g, b_g.reshape(1, D), W0, b0.reshape(1, -1),
                      W1, b1.reshape(1, -1), W2, b2.reshape(1, -1))


# speedup vs baseline: 28.6544x; 1.6877x over previous
import functools

import jax
import jax.numpy as jnp
from jax import lax
from jax.experimental import pallas as pl
from jax.experimental.pallas import tpu as pltpu
from jax.experimental.pallas import tpu_sc as plsc

F32 = jnp.float32

N = 10000
E = 320000
D = 128
G = 64
HD = D // 2

NC = 2
NS = 16
NW = NC * NS
EPW = E // NW
DEGR = (N + 127) // 128

CH = 125
EPS = E // NS
NCH = EPS // CH
ZB = 80
NZ = N // ZB


def _build_deg(interpret=False):
    mesh = plsc.VectorSubcoreMesh(core_axis_name="c", subcore_axis_name="s")

    @functools.partial(
        pl.kernel,
        out_type=jax.ShapeDtypeStruct((NC, DEGR, 128), F32),
        mesh=mesh,
        interpret=interpret,
        compiler_params=pltpu.CompilerParams(needs_layout_passes=False, use_tc_tiling_on_sc=False),
        scratch_types=[
            pltpu.VMEM((EPW,), jnp.int32),
            pltpu.VMEM((1, DEGR), jnp.int32),
            pltpu.VMEM((DEGR * 128,), F32),
            pltpu.VMEM((DEGR, 128), F32),
            pltpu.VMEM_SHARED((DEGR, 128), F32),
        ],
    )
    def deg_k(dst_hbm, rowidx_hbm, out_hbm, dst_v, rowidx_v, deg_flat,
              deg_priv, deg_sh):
        cid = lax.axis_index("c")
        sid = lax.axis_index("s")
        wid = sid * NC + cid

        zero16 = jnp.zeros((16,), F32)

        def zbody2(i, _):
            deg_flat[pl.ds(i * 16, 16)] = zero16
            return 0

        def zbody3(i, _):
            for j in range(128 // 16):
                deg_priv[i, pl.ds(j * 16, 16)] = zero16
            return 0

        lax.fori_loop(0, DEGR * 8, zbody2, 0)
        lax.fori_loop(0, DEGR, zbody3, 0)

        @pl.when(sid == 0)
        def _():
            pltpu.sync_copy(deg_priv, deg_sh)

        plsc.subcore_barrier()

        pltpu.sync_copy(dst_hbm.at[wid], dst_v)
        pltpu.sync_copy(rowidx_hbm, rowidx_v)

        ones16 = jnp.ones((16,), F32)

        def body(i, _):
            d = dst_v[pl.ds(i * 16, 16)]
            plsc.addupdate_scatter(deg_flat, [d], ones16)
            return 0

        lax.fori_loop(0, EPW // 16, body, 0)

        def pack(i, _):
            for j in range(128 // 16):
                deg_priv[i, pl.ds(j * 16, 16)] = deg_flat[
                    pl.ds(i * 128 + j * 16, 16)]
            return 0

        lax.fori_loop(0, DEGR, pack, 0)

        pltpu.sync_copy(deg_priv, deg_sh.at[rowidx_v.at[0]], add=True)

        plsc.subcore_barrier()

        @pl.when(sid == 0)
        def _():
            pltpu.sync_copy(deg_sh, out_hbm.at[cid])

    return deg_k


def _normxn_body(deg_ref, x_ref, xn_ref, norm_ref):
    d2 = deg_ref[...]
    deg = d2[0] + d2[1]
    norm = jnp.where(deg > 0, lax.rsqrt(jnp.maximum(deg, 1.0)), 0.0)
    norm_ref[...] = norm
    xn = x_ref[...] * norm
    xn_ref[0] = xn[:, :HD]
    xn_ref[1] = xn[:, HD:]


def _normxn_call(deg2, x, interpret=False):
    B = 2000
    grid = N // B
    return pl.pallas_call(
        _normxn_body,
        grid=(grid,),
        in_specs=[
            pl.BlockSpec((2, B, 1), lambda i: (0, i, 0)),
            pl.BlockSpec((B, D), lambda i: (i, 0)),
        ],
        out_specs=[
            pl.BlockSpec((2, B, HD), lambda i: (0, i, 0)),
            pl.BlockSpec((B, 1), lambda i: (i, 0)),
        ],
        out_shape=[
            jax.ShapeDtypeStruct((NC, N, HD), F32),
            jax.ShapeDtypeStruct((N, 1), F32),
        ],
        interpret=interpret,
    )(deg2, x)


def _build_agg(interpret=False):
    mesh = plsc.VectorSubcoreMesh(core_axis_name="c", subcore_axis_name="s")

    @functools.partial(
        pl.kernel,
        out_type=jax.ShapeDtypeStruct((NC, N, HD), F32),
        mesh=mesh,
        interpret=interpret,
        compiler_params=pltpu.CompilerParams(needs_layout_passes=False, use_tc_tiling_on_sc=False),
        scratch_types=[
            pltpu.VMEM((NCH, CH), jnp.int32),
            pltpu.VMEM((NCH, CH), jnp.int32),
            pltpu.VMEM((CH, HD), F32),
            pltpu.VMEM((CH, HD), F32),
            pltpu.VMEM_SHARED((N, HD), F32),
            pltpu.SemaphoreType.DMA,
            pltpu.SemaphoreType.DMA,
        ],
    )
    def agg_k(xn_hbm, src_hbm, dst_hbm, out_hbm,
              src_v, dst_v, rows0, rows1, acc_sh, sem0, sem1):
        cid = lax.axis_index("c")
        sid = lax.axis_index("s")
        tab = xn_hbm.at[cid]

        zero16 = jnp.zeros((16,), F32)

        def zbody(i, _):
            for j in range(HD // 16):
                rows0[i, pl.ds(j * 16, 16)] = zero16
            return 0

        lax.fori_loop(0, ZB, zbody, 0)

        for k in range((NZ + NS - 1) // NS):
            c = sid + NS * k

            @pl.when(c < NZ)
            def _(c=c):
                off = pl.multiple_of(c * ZB, ZB)
                pltpu.sync_copy(rows0.at[pl.ds(0, ZB)],
                                acc_sh.at[pl.ds(off, ZB)])

        plsc.subcore_barrier()

        pltpu.sync_copy(src_hbm.at[sid], src_v)
        pltpu.sync_copy(dst_hbm.at[sid], dst_v)

        pltpu.async_copy(tab.at[src_v.at[0]], rows0, sem0)

        def body(c2, _):
            c = 2 * c2
            pltpu.async_copy(tab.at[src_v.at[c + 1]], rows1, sem1)
            pltpu.make_async_copy(tab.at[src_v.at[c]], rows0, sem0).wait()
            pltpu.sync_copy(rows0, acc_sh.at[dst_v.at[c]], add=True)

            @pl.when(c2 + 1 < NCH // 2)
            def _():
                pltpu.async_copy(tab.at[src_v.at[c + 2]], rows0, sem0)

            pltpu.make_async_copy(tab.at[src_v.at[c + 1]], rows1, sem1).wait()
            pltpu.sync_copy(rows1, acc_sh.at[dst_v.at[c + 1]], add=True)
            return 0

        lax.fori_loop(0, NCH // 2, body, 0)

        plsc.subcore_barrier()

        for k in range((NZ + NS - 1) // NS):
            c = sid + NS * k

            @pl.when(c < NZ)
            def _(c=c):
                off = pl.multiple_of(c * ZB, ZB)
                pltpu.sync_copy(acc_sh.at[pl.ds(off, ZB)],
                                out_hbm.at[cid, pl.ds(off, ZB)])

        plsc.subcore_barrier()

    return agg_k


_DB = 2000


def _elu(v):
    return jnp.where(v > 0, v, jnp.exp(v) - 1.0)


def _head_body(p_ref, norm_ref, batch_ref, Wg_ref, bg_ref,
               W0_ref, b0_ref, W1_ref, b1_ref, W2_ref, b2_ref, out_ref,
               acc_s, acc_m, acc_c, carry_v, carry_b):
    B = _DB
    i = pl.program_id(0)
    nsteps = pl.num_programs(0)

    @pl.when(i == 0)
    def _():
        acc_s[...] = jnp.zeros_like(acc_s)
        acc_m[...] = jnp.full_like(acc_m, -1e30)
        acc_c[...] = jnp.zeros_like(acc_c)
        carry_v[...] = jnp.full_like(carry_v, -1e30)
        carry_b[...] = jnp.full_like(carry_b, -1)

    p = p_ref[...]
    agg = jnp.concatenate([p[0], p[1]],
                          axis=1) * norm_ref[...]
    h = _elu(agg @ Wg_ref[...] + bg_ref[...])

    bt = batch_ref[...]
    onehot = (bt == lax.broadcasted_iota(jnp.int32, (1, G), 1)).astype(F32)

    dn = (((0,), (0,)), ((), ()))
    acc_s[...] += lax.dot_general(onehot, h, dn, preferred_element_type=F32)
    acc_c[...] += lax.dot_general(onehot, jnp.ones((B, 1), F32), dn,
                                  preferred_element_type=F32)

    pm = jnp.where(bt == carry_b[...], jnp.maximum(h, carry_v[...]), h)
    sh = 1
    while sh < B:
        pm_s = jnp.concatenate(
            [jnp.full((sh, D), -1e30, F32), pm[:B - sh]], axis=0)
        bt_s = jnp.concatenate(
            [jnp.full((sh, 1), -1, jnp.int32), bt[:B - sh]], axis=0)
        pm = jnp.where(bt_s == bt, jnp.maximum(pm, pm_s), pm)
        sh *= 2

    bt_n = jnp.concatenate(
        [bt[1:], jnp.full((1, 1), -2, jnp.int32)], axis=0)
    endm = (bt != bt_n).astype(F32)
    m_part = lax.dot_general(onehot, pm * endm, dn, preferred_element_type=F32)
    g_part = lax.dot_general(onehot, endm, dn, preferred_element_type=F32)
    acc_m[...] = jnp.where(g_part > 0, jnp.maximum(acc_m[...], m_part),
                           acc_m[...])
    carry_v[...] = pm[B - 1:B, :]
    carry_b[...] = bt[B - 1:B, :]

    @pl.when(i == nsteps - 1)
    def _():
        cnt = acc_c[...]
        mean = acc_s[...] / jnp.maximum(cnt, 1.0)
        mx = jnp.where(cnt > 0, acc_m[...], 0.0)
        r = jnp.concatenate([mean, mx, acc_s[...]], axis=1)
        y = _elu(r @ W0_ref[...] + b0_ref[...])
        y = _elu(y @ W1_ref[...] + b1_ref[...])
        out_ref[...] = y @ W2_ref[...] + b2_ref[...]


def _head_call(p, norm, batch2, W_g, b_g, W0, b0, W1, b1, W2, b2,
               interpret=False):
    B = _DB
    grid = N // B
    OUT_CH = W2.shape[1]

    def full(shape):
        return pl.BlockSpec(shape, lambda *_: tuple(0 for _ in shape))

    return pl.pallas_call(
        _head_body,
        grid=(grid,),
        in_specs=[
            pl.BlockSpec((2, B, HD), lambda i: (0, i, 0)),
            pl.BlockSpec((B, 1), lambda i: (i, 0)),
            pl.BlockSpec((B, 1), lambda i: (i, 0)),
            full((D, D)), full((1, D)),
            full((3 * D, W0.shape[1])), full((1, W0.shape[1])),
            full((W1.shape[0], W1.shape[1])), full((1, W1.shape[1])),
            full((W2.shape[0], OUT_CH)), full((1, OUT_CH)),
        ],
        out_specs=pl.BlockSpec((G, OUT_CH), lambda i: (0, 0)),
        out_shape=jax.ShapeDtypeStruct((G, OUT_CH), F32),
        scratch_shapes=[
            pltpu.VMEM((G, D), F32),
            pltpu.VMEM((G, D), F32),
            pltpu.VMEM((G, 1), F32),
            pltpu.VMEM((1, D), F32),
            pltpu.VMEM((1, 1), jnp.int32),
        ],
        interpret=interpret,
    )(p, norm, batch2, W_g, b_g, W0, b0, W1, b1, W2, b2)


def kernel(x, edge_index, batch, W_g, b_g, W0, b0, W1, b1, W2, b2):
    src = edge_index[0]
    dst = edge_index[1]

    deg2 = _build_deg()(dst.reshape(NW, EPW),
                        jnp.arange(DEGR, dtype=jnp.int32).reshape(1, DEGR))
    deg2 = deg2.reshape(2, DEGR * 128, 1)[:, :N]

    xn2, norm = _normxn_call(deg2, x)

    p = _build_agg()(xn2, src.reshape(NS, NCH, CH), dst.reshape(NS, NCH, CH))

    return _head_call(p, norm, batch.reshape(N, 1),
                      W_g, b_g.reshape(1, D), W0, b0.reshape(1, -1),
                      W1, b1.reshape(1, -1), W2, b2.reshape(1, -1))


# speedup vs baseline: 32.8439x; 1.1462x over previous
import functools

import jax
import jax.numpy as jnp
from jax import lax
from jax.experimental import pallas as pl
from jax.experimental.pallas import tpu as pltpu
from jax.experimental.pallas import tpu_sc as plsc

F32 = jnp.float32

N = 10000
E = 320000
D = 128
G = 64
HD = D // 2

NC = 2
NS = 16
NW = NC * NS
EPW = E // NW
DEGR = (N + 127) // 128

CH = 125
EPS = E // NS
NCH = EPS // CH
ZB = 80
NZ = N // ZB


def _build_deg(interpret=False):
    mesh = plsc.VectorSubcoreMesh(core_axis_name="c", subcore_axis_name="s")

    @functools.partial(
        pl.kernel,
        out_type=jax.ShapeDtypeStruct((NC, DEGR, 128), F32),
        mesh=mesh,
        interpret=interpret,
        compiler_params=pltpu.CompilerParams(needs_layout_passes=False, use_tc_tiling_on_sc=False),
        scratch_types=[
            pltpu.VMEM((EPW,), jnp.int32),
            pltpu.VMEM((1, DEGR), jnp.int32),
            pltpu.VMEM((DEGR * 128,), F32),
            pltpu.VMEM((DEGR, 128), F32),
            pltpu.VMEM_SHARED((DEGR, 128), F32),
        ],
    )
    def deg_k(dst_hbm, rowidx_hbm, out_hbm, dst_v, rowidx_v, deg_flat,
              deg_priv, deg_sh):
        cid = lax.axis_index("c")
        sid = lax.axis_index("s")
        wid = sid * NC + cid

        zero16 = jnp.zeros((16,), F32)

        def zbody2(i, _):
            deg_flat[pl.ds(i * 16, 16)] = zero16
            return 0

        def zbody3(i, _):
            for j in range(128 // 16):
                deg_priv[i, pl.ds(j * 16, 16)] = zero16
            return 0

        lax.fori_loop(0, DEGR * 8, zbody2, 0)
        lax.fori_loop(0, DEGR, zbody3, 0)

        @pl.when(sid == 0)
        def _():
            pltpu.sync_copy(deg_priv, deg_sh)

        plsc.subcore_barrier()

        pltpu.sync_copy(dst_hbm.at[wid], dst_v)
        pltpu.sync_copy(rowidx_hbm, rowidx_v)

        ones16 = jnp.ones((16,), F32)

        def body(i, _):
            d = dst_v[pl.ds(i * 16, 16)]
            plsc.addupdate_scatter(deg_flat, [d], ones16)
            return 0

        lax.fori_loop(0, EPW // 16, body, 0)

        def pack(i, _):
            for j in range(128 // 16):
                deg_priv[i, pl.ds(j * 16, 16)] = deg_flat[
                    pl.ds(i * 128 + j * 16, 16)]
            return 0

        lax.fori_loop(0, DEGR, pack, 0)

        pltpu.sync_copy(deg_priv, deg_sh.at[rowidx_v.at[0]], add=True)

        plsc.subcore_barrier()

        @pl.when(sid == 0)
        def _():
            pltpu.sync_copy(deg_sh, out_hbm.at[cid])

    return deg_k


def _normxn_body(deg_ref, x_ref, xn_ref, norm_ref):
    d2 = deg_ref[...]
    deg = d2[0] + d2[1]
    norm = jnp.where(deg > 0, lax.rsqrt(jnp.maximum(deg, 1.0)), 0.0)
    norm_ref[...] = norm
    xn = x_ref[...] * norm
    xn_ref[0] = xn[:, :HD]
    xn_ref[1] = xn[:, HD:]


def _normxn_call(deg2, x, interpret=False):
    B = 2000
    grid = N // B
    return pl.pallas_call(
        _normxn_body,
        grid=(grid,),
        in_specs=[
            pl.BlockSpec((2, B, 1), lambda i: (0, i, 0)),
            pl.BlockSpec((B, D), lambda i: (i, 0)),
        ],
        out_specs=[
            pl.BlockSpec((2, B, HD), lambda i: (0, i, 0)),
            pl.BlockSpec((B, 1), lambda i: (i, 0)),
        ],
        out_shape=[
            jax.ShapeDtypeStruct((NC, N, HD), F32),
            jax.ShapeDtypeStruct((N, 1), F32),
        ],
        interpret=interpret,
    )(deg2, x)


def _build_agg(interpret=False):
    mesh = plsc.VectorSubcoreMesh(core_axis_name="c", subcore_axis_name="s")

    @functools.partial(
        pl.kernel,
        out_type=jax.ShapeDtypeStruct((NC, N, HD), F32),
        mesh=mesh,
        interpret=interpret,
        compiler_params=pltpu.CompilerParams(needs_layout_passes=False, use_tc_tiling_on_sc=False),
        scratch_types=[
            pltpu.VMEM((NCH, CH), jnp.int32),
            pltpu.VMEM((NCH, CH), jnp.int32),
            pltpu.VMEM((CH, HD), F32),
            pltpu.VMEM((CH, HD), F32),
            pltpu.VMEM((CH, HD), F32),
            pltpu.VMEM((CH, HD), F32),
            pltpu.VMEM_SHARED((N, HD), F32),
            pltpu.SemaphoreType.DMA,
            pltpu.SemaphoreType.DMA,
            pltpu.SemaphoreType.DMA,
            pltpu.SemaphoreType.DMA,
        ],
    )
    def agg_k(xn_hbm, src_hbm, dst_hbm, out_hbm,
              src_v, dst_v, rows0, rows1, rows2, rows3, acc_sh,
              sem0, sem1, sem2, sem3):
        cid = lax.axis_index("c")
        sid = lax.axis_index("s")
        tab = xn_hbm.at[cid]

        zero16 = jnp.zeros((16,), F32)

        def zbody(i, _):
            for j in range(HD // 16):
                rows0[i, pl.ds(j * 16, 16)] = zero16
            return 0

        lax.fori_loop(0, ZB, zbody, 0)

        for k in range((NZ + NS - 1) // NS):
            c = sid + NS * k

            @pl.when(c < NZ)
            def _(c=c):
                off = pl.multiple_of(c * ZB, ZB)
                pltpu.sync_copy(rows0.at[pl.ds(0, ZB)],
                                acc_sh.at[pl.ds(off, ZB)])

        plsc.subcore_barrier()

        pltpu.sync_copy(src_hbm.at[sid], src_v)
        pltpu.sync_copy(dst_hbm.at[sid], dst_v)

        rows = (rows0, rows1, rows2, rows3)
        sems = (sem0, sem1, sem2, sem3)
        NB = 4
        for b in range(NB - 1):
            pltpu.async_copy(tab.at[src_v.at[b]], rows[b], sems[b])

        def body(c4, _):
            for b in range(NB):
                c = NB * c4 + b

                @pl.when(c + NB - 1 < NCH)
                def _(c=c, b=b):
                    pltpu.async_copy(tab.at[src_v.at[c + NB - 1]],
                                     rows[(b + NB - 1) % NB],
                                     sems[(b + NB - 1) % NB])

                pltpu.make_async_copy(tab.at[src_v.at[c]], rows[b],
                                      sems[b]).wait()
                pltpu.sync_copy(rows[b], acc_sh.at[dst_v.at[c]], add=True)
            return 0

        lax.fori_loop(0, NCH // NB, body, 0)

        plsc.subcore_barrier()

        for k in range((NZ + NS - 1) // NS):
            c = sid + NS * k

            @pl.when(c < NZ)
            def _(c=c):
                off = pl.multiple_of(c * ZB, ZB)
                pltpu.sync_copy(acc_sh.at[pl.ds(off, ZB)],
                                out_hbm.at[cid, pl.ds(off, ZB)])

        plsc.subcore_barrier()

    return agg_k


_DB = 2000


def _elu(v):
    return jnp.where(v > 0, v, jnp.exp(v) - 1.0)


def _head_body(p_ref, norm_ref, batch_ref, Wg_ref, bg_ref,
               W0_ref, b0_ref, W1_ref, b1_ref, W2_ref, b2_ref, out_ref,
               acc_s, acc_m, acc_c, carry_v, carry_b):
    B = _DB
    i = pl.program_id(0)
    nsteps = pl.num_programs(0)

    @pl.when(i == 0)
    def _():
        acc_s[...] = jnp.zeros_like(acc_s)
        acc_m[...] = jnp.full_like(acc_m, -1e30)
        acc_c[...] = jnp.zeros_like(acc_c)
        carry_v[...] = jnp.full_like(carry_v, -1e30)
        carry_b[...] = jnp.full_like(carry_b, -1)

    p = p_ref[...]
    agg = jnp.concatenate([p[0], p[1]],
                          axis=1) * norm_ref[...]
    h = _elu(agg @ Wg_ref[...] + bg_ref[...])

    bt = batch_ref[...]
    onehot = (bt == lax.broadcasted_iota(jnp.int32, (1, G), 1)).astype(F32)

    dn = (((0,), (0,)), ((), ()))
    acc_s[...] += lax.dot_general(onehot, h, dn, preferred_element_type=F32)
    acc_c[...] += lax.dot_general(onehot, jnp.ones((B, 1), F32), dn,
                                  preferred_element_type=F32)

    pm = jnp.where(bt == carry_b[...], jnp.maximum(h, carry_v[...]), h)
    sh = 1
    while sh < B:
        pm_s = jnp.concatenate(
            [jnp.full((sh, D), -1e30, F32), pm[:B - sh]], axis=0)
        bt_s = jnp.concatenate(
            [jnp.full((sh, 1), -1, jnp.int32), bt[:B - sh]], axis=0)
        pm = jnp.where(bt_s == bt, jnp.maximum(pm, pm_s), pm)
        sh *= 2

    bt_n = jnp.concatenate(
        [bt[1:], jnp.full((1, 1), -2, jnp.int32)], axis=0)
    endm = (bt != bt_n).astype(F32)
    m_part = lax.dot_general(onehot, pm * endm, dn, preferred_element_type=F32)
    g_part = lax.dot_general(onehot, endm, dn, preferred_element_type=F32)
    acc_m[...] = jnp.where(g_part > 0, jnp.maximum(acc_m[...], m_part),
                           acc_m[...])
    carry_v[...] = pm[B - 1:B, :]
    carry_b[...] = bt[B - 1:B, :]

    @pl.when(i == nsteps - 1)
    def _():
        cnt = acc_c[...]
        mean = acc_s[...] / jnp.maximum(cnt, 1.0)
        mx = jnp.where(cnt > 0, acc_m[...], 0.0)
        r = jnp.concatenate([mean, mx, acc_s[...]], axis=1)
        y = _elu(r @ W0_ref[...] + b0_ref[...])
        y = _elu(y @ W1_ref[...] + b1_ref[...])
        out_ref[...] = y @ W2_ref[...] + b2_ref[...]


def _head_call(p, norm, batch2, W_g, b_g, W0, b0, W1, b1, W2, b2,
               interpret=False):
    B = _DB
    grid = N // B
    OUT_CH = W2.shape[1]

    def full(shape):
        return pl.BlockSpec(shape, lambda *_: tuple(0 for _ in shape))

    return pl.pallas_call(
        _head_body,
        grid=(grid,),
        in_specs=[
            pl.BlockSpec((2, B, HD), lambda i: (0, i, 0)),
            pl.BlockSpec((B, 1), lambda i: (i, 0)),
            pl.BlockSpec((B, 1), lambda i: (i, 0)),
            full((D, D)), full((1, D)),
            full((3 * D, W0.shape[1])), full((1, W0.shape[1])),
            full((W1.shape[0], W1.shape[1])), full((1, W1.shape[1])),
            full((W2.shape[0], OUT_CH)), full((1, OUT_CH)),
        ],
        out_specs=pl.BlockSpec((G, OUT_CH), lambda i: (0, 0)),
        out_shape=jax.ShapeDtypeStruct((G, OUT_CH), F32),
        scratch_shapes=[
            pltpu.VMEM((G, D), F32),
            pltpu.VMEM((G, D), F32),
            pltpu.VMEM((G, 1), F32),
            pltpu.VMEM((1, D), F32),
            pltpu.VMEM((1, 1), jnp.int32),
        ],
        interpret=interpret,
    )(p, norm, batch2, W_g, b_g, W0, b0, W1, b1, W2, b2)


def kernel(x, edge_index, batch, W_g, b_g, W0, b0, W1, b1, W2, b2):
    src = edge_index[0]
    dst = edge_index[1]

    deg2 = _build_deg()(dst.reshape(NW, EPW),
                        jnp.arange(DEGR, dtype=jnp.int32).reshape(1, DEGR))
    deg2 = deg2.reshape(2, DEGR * 128, 1)[:, :N]

    xn2, norm = _normxn_call(deg2, x)

    p = _build_agg()(xn2, src.reshape(NS, NCH, CH), dst.reshape(NS, NCH, CH))

    return _head_call(p, norm, batch.reshape(N, 1),
                      W_g, b_g.reshape(1, D), W0, b0.reshape(1, -1),
                      W1, b1.reshape(1, -1), W2, b2.reshape(1, -1))


# speedup vs baseline: 33.7910x; 1.0288x over previous
import functools

import jax
import jax.numpy as jnp
from jax import lax
from jax.experimental import pallas as pl
from jax.experimental.pallas import tpu as pltpu
from jax.experimental.pallas import tpu_sc as plsc

F32 = jnp.float32

N = 10000
E = 320000
D = 128
G = 64
HD = D // 2

NC = 2
NS = 16
NW = NC * NS
EPW = E // NW
DEGR = (N + 127) // 128

CH = 125
EPS = E // NS
NCH = EPS // CH
ZB = 80
NZ = N // ZB


def _build_deg(interpret=False):
    mesh = plsc.VectorSubcoreMesh(core_axis_name="c", subcore_axis_name="s")

    @functools.partial(
        pl.kernel,
        out_type=jax.ShapeDtypeStruct((NC, DEGR, 128), F32),
        mesh=mesh,
        interpret=interpret,
        compiler_params=pltpu.CompilerParams(needs_layout_passes=False, use_tc_tiling_on_sc=False),
        scratch_types=[
            pltpu.VMEM((EPW,), jnp.int32),
            pltpu.VMEM((1, DEGR), jnp.int32),
            pltpu.VMEM((DEGR * 128,), F32),
            pltpu.VMEM((DEGR, 128), F32),
            pltpu.VMEM_SHARED((DEGR, 128), F32),
        ],
    )
    def deg_k(dst_hbm, rowidx_hbm, out_hbm, dst_v, rowidx_v, deg_flat,
              deg_priv, deg_sh):
        cid = lax.axis_index("c")
        sid = lax.axis_index("s")
        wid = sid * NC + cid

        zero16 = jnp.zeros((16,), F32)

        def zbody2(i, _):
            deg_flat[pl.ds(i * 16, 16)] = zero16
            return 0

        def zbody3(i, _):
            for j in range(128 // 16):
                deg_priv[i, pl.ds(j * 16, 16)] = zero16
            return 0

        lax.fori_loop(0, DEGR * 8, zbody2, 0)
        lax.fori_loop(0, DEGR, zbody3, 0)

        @pl.when(sid == 0)
        def _():
            pltpu.sync_copy(deg_priv, deg_sh)

        plsc.subcore_barrier()

        pltpu.sync_copy(dst_hbm.at[wid], dst_v)
        pltpu.sync_copy(rowidx_hbm, rowidx_v)

        ones16 = jnp.ones((16,), F32)

        def body(i, _):
            d = dst_v[pl.ds(i * 16, 16)]
            plsc.addupdate_scatter(deg_flat, [d], ones16)
            return 0

        lax.fori_loop(0, EPW // 16, body, 0)

        def pack(i, _):
            for j in range(128 // 16):
                deg_priv[i, pl.ds(j * 16, 16)] = deg_flat[
                    pl.ds(i * 128 + j * 16, 16)]
            return 0

        lax.fori_loop(0, DEGR, pack, 0)

        pltpu.sync_copy(deg_priv, deg_sh.at[rowidx_v.at[0]], add=True)

        plsc.subcore_barrier()

        @pl.when(sid == 0)
        def _():
            pltpu.sync_copy(deg_sh, out_hbm.at[cid])

    return deg_k


def _normxn_body(deg_ref, x_ref, xn_ref, norm_ref):
    d2 = deg_ref[...]
    deg = d2[0] + d2[1]
    norm = jnp.where(deg > 0, lax.rsqrt(jnp.maximum(deg, 1.0)), 0.0)
    norm_ref[...] = norm
    xn = x_ref[...] * norm
    xn_ref[0] = xn[:, :HD]
    xn_ref[1] = xn[:, HD:]


def _normxn_call(deg2, x, interpret=False):
    B = 2000
    grid = N // B
    return pl.pallas_call(
        _normxn_body,
        grid=(grid,),
        in_specs=[
            pl.BlockSpec((2, B, 1), lambda i: (0, i, 0)),
            pl.BlockSpec((B, D), lambda i: (i, 0)),
        ],
        out_specs=[
            pl.BlockSpec((2, B, HD), lambda i: (0, i, 0)),
            pl.BlockSpec((B, 1), lambda i: (i, 0)),
        ],
        out_shape=[
            jax.ShapeDtypeStruct((NC, N, HD), F32),
            jax.ShapeDtypeStruct((N, 1), F32),
        ],
        interpret=interpret,
    )(deg2, x)


def _build_agg(interpret=False):
    mesh = plsc.VectorSubcoreMesh(core_axis_name="c", subcore_axis_name="s")

    @functools.partial(
        pl.kernel,
        out_type=jax.ShapeDtypeStruct((NC, N, HD), F32),
        mesh=mesh,
        interpret=interpret,
        compiler_params=pltpu.CompilerParams(needs_layout_passes=False, use_tc_tiling_on_sc=False),
        scratch_types=[
            pltpu.VMEM((NCH, CH), jnp.int32),
            pltpu.VMEM((NCH, CH), jnp.int32),
            pltpu.VMEM((CH, HD), F32),
            pltpu.VMEM((CH, HD), F32),
            pltpu.VMEM((CH, HD), F32),
            pltpu.VMEM((CH, HD), F32),
            pltpu.VMEM_SHARED((N, HD), F32),
            pltpu.SemaphoreType.DMA,
            pltpu.SemaphoreType.DMA,
            pltpu.SemaphoreType.DMA,
            pltpu.SemaphoreType.DMA,
        ],
    )
    def agg_k(xn_hbm, src_hbm, dst_hbm, out_hbm,
              src_v, dst_v, rows0, rows1, rows2, rows3, acc_sh,
              sem0, sem1, sem2, sem3):
        cid = lax.axis_index("c")
        sid = lax.axis_index("s")
        tab = xn_hbm.at[cid]

        zero16 = jnp.zeros((16,), F32)

        def zbody(i, _):
            for j in range(HD // 16):
                rows0[i, pl.ds(j * 16, 16)] = zero16
            return 0

        lax.fori_loop(0, ZB, zbody, 0)

        for k in range((NZ + NS - 1) // NS):
            c = sid + NS * k

            @pl.when(c < NZ)
            def _(c=c):
                off = pl.multiple_of(c * ZB, ZB)
                pltpu.sync_copy(rows0.at[pl.ds(0, ZB)],
                                acc_sh.at[pl.ds(off, ZB)])

        plsc.subcore_barrier()

        pltpu.sync_copy(src_hbm.at[sid], src_v)
        pltpu.sync_copy(dst_hbm.at[sid], dst_v)

        rows = (rows0, rows1, rows2, rows3)
        sems = (sem0, sem1, sem2, sem3)
        NB = 4
        for b in range(NB - 1):
            pltpu.async_copy(tab.at[src_v.at[b]], rows[b], sems[b])

        def body(c4, _):
            for b in range(NB):
                c = NB * c4 + b

                @pl.when(c + NB - 1 < NCH)
                def _(c=c, b=b):
                    pltpu.async_copy(tab.at[src_v.at[c + NB - 1]],
                                     rows[(b + NB - 1) % NB],
                                     sems[(b + NB - 1) % NB])

                pltpu.make_async_copy(tab.at[src_v.at[c]], rows[b],
                                      sems[b]).wait()
                pltpu.sync_copy(rows[b], acc_sh.at[dst_v.at[c]], add=True)
            return 0

        lax.fori_loop(0, NCH // NB, body, 0)

        plsc.subcore_barrier()

        for k in range((NZ + NS - 1) // NS):
            c = sid + NS * k

            @pl.when(c < NZ)
            def _(c=c):
                off = pl.multiple_of(c * ZB, ZB)
                pltpu.sync_copy(acc_sh.at[pl.ds(off, ZB)],
                                out_hbm.at[cid, pl.ds(off, ZB)])

        plsc.subcore_barrier()

    return agg_k


_DB = 2000


def _elu(v):
    return jnp.where(v > 0, v, jnp.exp(v) - 1.0)


def _head_body(p_ref, norm_ref, batch_ref, Wg_ref, bg_ref,
               W0_ref, b0_ref, W1_ref, b1_ref, W2_ref, b2_ref, out_ref,
               acc_s, acc_m, acc_c, carry_v, carry_b):
    B = _DB
    i = pl.program_id(0)
    nsteps = pl.num_programs(0)

    @pl.when(i == 0)
    def _():
        acc_s[...] = jnp.zeros_like(acc_s)
        acc_m[...] = jnp.full_like(acc_m, -1e30)
        acc_c[...] = jnp.zeros_like(acc_c)
        carry_v[...] = jnp.full_like(carry_v, -1e30)
        carry_b[...] = jnp.full_like(carry_b, -1)

    p = p_ref[...]
    agg = jnp.concatenate([p[0], p[1]],
                          axis=1) * norm_ref[...]
    h = _elu(agg @ Wg_ref[...] + bg_ref[...])

    bt = batch_ref[...]
    onehot = (bt == lax.broadcasted_iota(jnp.int32, (1, G), 1)).astype(F32)

    dn = (((0,), (0,)), ((), ()))
    acc_s[...] += lax.dot_general(onehot, h, dn, preferred_element_type=F32)
    acc_c[...] += lax.dot_general(onehot, jnp.ones((B, 1), F32), dn,
                                  preferred_element_type=F32)

    R = 8
    NG = B // R
    pm = jnp.where(bt == carry_b[...], jnp.maximum(h, carry_v[...]), h)
    pm3 = pm.reshape(NG, R, D)
    bt3 = bt.reshape(NG, R, 1)
    sh = 1
    while sh < R:
        pm_s = jnp.concatenate(
            [jnp.full((NG, sh, D), -1e30, F32), pm3[:, :R - sh]], axis=1)
        bt_s = jnp.concatenate(
            [jnp.full((NG, sh, 1), -1, jnp.int32), bt3[:, :R - sh]], axis=1)
        pm3 = jnp.where(bt_s == bt3, jnp.maximum(pm3, pm_s), pm3)
        sh *= 2
    gt = pm3[:, R - 1]
    gb = bt3[:, R - 1]
    sh = 1
    while sh < NG:
        gt_s = jnp.concatenate(
            [jnp.full((sh, D), -1e30, F32), gt[:NG - sh]], axis=0)
        gb_s = jnp.concatenate(
            [jnp.full((sh, 1), -1, jnp.int32), gb[:NG - sh]], axis=0)
        gt = jnp.where(gb_s == gb, jnp.maximum(gt, gt_s), gt)
        sh *= 2
    prev = jnp.concatenate(
        [jnp.full((1, D), -1e30, F32), gt[:NG - 1]], axis=0)
    prevb = jnp.concatenate(
        [jnp.full((1, 1), -1, jnp.int32), gb[:NG - 1]], axis=0)
    pm3 = jnp.where(bt3 == prevb[:, None, :],
                    jnp.maximum(pm3, prev[:, None, :]), pm3)
    pm = pm3.reshape(B, D)

    bt_n = jnp.concatenate(
        [bt[1:], jnp.full((1, 1), -2, jnp.int32)], axis=0)
    endm = (bt != bt_n).astype(F32)
    m_part = lax.dot_general(onehot, pm * endm, dn, preferred_element_type=F32)
    g_part = lax.dot_general(onehot, endm, dn, preferred_element_type=F32)
    acc_m[...] = jnp.where(g_part > 0, jnp.maximum(acc_m[...], m_part),
                           acc_m[...])
    carry_v[...] = pm[B - 1:B, :]
    carry_b[...] = bt[B - 1:B, :]

    @pl.when(i == nsteps - 1)
    def _():
        cnt = acc_c[...]
        mean = acc_s[...] / jnp.maximum(cnt, 1.0)
        mx = jnp.where(cnt > 0, acc_m[...], 0.0)
        r = jnp.concatenate([mean, mx, acc_s[...]], axis=1)
        y = _elu(r @ W0_ref[...] + b0_ref[...])
        y = _elu(y @ W1_ref[...] + b1_ref[...])
        out_ref[...] = y @ W2_ref[...] + b2_ref[...]


def _head_call(p, norm, batch2, W_g, b_g, W0, b0, W1, b1, W2, b2,
               interpret=False):
    B = _DB
    grid = N // B
    OUT_CH = W2.shape[1]

    def full(shape):
        return pl.BlockSpec(shape, lambda *_: tuple(0 for _ in shape))

    return pl.pallas_call(
        _head_body,
        grid=(grid,),
        in_specs=[
            pl.BlockSpec((2, B, HD), lambda i: (0, i, 0)),
            pl.BlockSpec((B, 1), lambda i: (i, 0)),
            pl.BlockSpec((B, 1), lambda i: (i, 0)),
            full((D, D)), full((1, D)),
            full((3 * D, W0.shape[1])), full((1, W0.shape[1])),
            full((W1.shape[0], W1.shape[1])), full((1, W1.shape[1])),
            full((W2.shape[0], OUT_CH)), full((1, OUT_CH)),
        ],
        out_specs=pl.BlockSpec((G, OUT_CH), lambda i: (0, 0)),
        out_shape=jax.ShapeDtypeStruct((G, OUT_CH), F32),
        scratch_shapes=[
            pltpu.VMEM((G, D), F32),
            pltpu.VMEM((G, D), F32),
            pltpu.VMEM((G, 1), F32),
            pltpu.VMEM((1, D), F32),
            pltpu.VMEM((1, 1), jnp.int32),
        ],
        interpret=interpret,
    )(p, norm, batch2, W_g, b_g, W0, b0, W1, b1, W2, b2)


def kernel(x, edge_index, batch, W_g, b_g, W0, b0, W1, b1, W2, b2):
    src = edge_index[0]
    dst = edge_index[1]

    deg2 = _build_deg()(dst.reshape(NW, EPW),
                        jnp.arange(DEGR, dtype=jnp.int32).reshape(1, DEGR))
    deg2 = deg2.reshape(2, DEGR * 128, 1)[:, :N]

    xn2, norm = _normxn_call(deg2, x)

    p = _build_agg()(xn2, src.reshape(NS, NCH, CH), dst.reshape(NS, NCH, CH))

    return _head_call(p, norm, batch.reshape(N, 1),
                      W_g, b_g.reshape(1, D), W0, b0.reshape(1, -1),
                      W1, b1.reshape(1, -1), W2, b2.reshape(1, -1))
